# Initial kernel scaffold; baseline (speedup 1.0000x reference)
import jax
import jax.numpy as jnp
from jax.experimental import pallas as pl


def kernel(feats, norm_points, kq_indices, kernel_offsets, Wp1, g1, b1, Wp2, g2, b2, Wp3, bp3, Wq, bq, Wv, bv, Wo, bo, pos_enc):
    raise NotImplementedError("write your pallas kernel here")


# speedup vs baseline: 65.8267x; 65.8267x over previous
import functools

import jax
import jax.numpy as jnp
from jax import lax
from jax.experimental import pallas as pl
from jax.experimental.pallas import tpu as pltpu
from jax.experimental.pallas import tpu_sc as plsc


def _mlp_t1(pts, w1):
    return (pts[:, 0:1] * w1[0, :][None, :]
            + pts[:, 1:2] * w1[1, :][None, :]
            + pts[:, 2:3] * w1[2, :][None, :])


def _mlp_t2(t, w2):
    return (t[:, 0:1] * w2[0, :][None, :]
            + t[:, 1:2] * w2[1, :][None, :]
            + t[:, 2:3] * w2[2, :][None, :])


def _stats_body(np_ref, Wp1, g1, b1, Wp2, pe_ref,
                m1_out, v1_out, m2_out, v2_out, P_out):
    pts = np_ref[...]
    t = _mlp_t1(pts, Wp1[...])
    m = jnp.mean(t, axis=0)
    var = jnp.mean((t - m[None, :]) ** 2, axis=0)
    m1_out[...] = m
    v1_out[...] = var
    t = (t - m[None, :]) / jnp.sqrt(var[None, :] + 1e-5) * g1[...][None, :] \
        + b1[...][None, :]
    t = jnp.maximum(t, 0.0)
    t2 = _mlp_t2(t, Wp2[...])
    m2 = jnp.mean(t2, axis=0)
    m2_out[...] = m2
    v2_out[...] = jnp.mean((t2 - m2[None, :]) ** 2, axis=0)
    pe = pe_ref[...]
    kv, _, ah = pe.shape
    c_ = pe.shape[1] * ah
    pn = jnp.sqrt(jnp.sum(pe * pe, axis=-1, keepdims=True))
    npe = pe / jnp.maximum(pn, 1e-12)
    pe_flat = npe.reshape(kv, c_)
    jj = lax.broadcasted_iota(jnp.int32, (c_, c_), 0) // ah
    jp = lax.broadcasted_iota(jnp.int32, (c_, c_), 1) // ah
    mask = (jj == jp).astype(jnp.float32)
    pe_t = pe_flat.T
    P = pe_t[:, :, None] * mask[:, None, :]
    P_out[...] = P.reshape(c_, kv * c_)


def _block_body(np_ref, feats_ref, m1, v1, g1, b1, m2, v2, g2, b2,
                Wp1, Wp2, Wp3, bp3, Wq, bq, Wv, bv, P_ref,
                abc_out, v_out):
    t = _mlp_t1(np_ref[...], Wp1[...])
    t = (t - m1[...][None, :]) / jnp.sqrt(v1[...][None, :] + 1e-5) \
        * g1[...][None, :] + b1[...][None, :]
    t = jnp.maximum(t, 0.0)
    t2 = _mlp_t2(t, Wp2[...])
    t2 = (t2 - m2[...][None, :]) / jnp.sqrt(v2[...][None, :] + 1e-5) \
        * g2[...][None, :] + b2[...][None, :]
    t2 = jnp.maximum(t2, 0.0)
    intra = jnp.dot(t2, Wp3[...], preferred_element_type=jnp.float32) \
        + bp3[...][None, :]
    x = feats_ref[...] + intra
    q = jnp.dot(x, Wq[...], preferred_element_type=jnp.float32) + bq[...][None, :]
    v_out[...] = jnp.dot(x, Wv[...], preferred_element_type=jnp.float32) \
        + bv[...][None, :]
    n_, c_ = q.shape
    ah = 16
    q3 = q.reshape(n_, c_ // ah, ah)
    qn = jnp.sqrt(jnp.sum(q3 * q3, axis=-1, keepdims=True))
    nq = (q3 / jnp.maximum(qn, 1e-12)).reshape(n_, c_)
    abc_out[...] = jnp.dot(nq, P_ref[...], preferred_element_type=jnp.float32)


def _make_sc_edge_kernel(N, E, C, KV):
    NC, NS = 2, 16
    NW = NC * NS
    EPW = E // NW
    B = 80
    NCHUNK = EPW // B
    ZB = 16
    NRB = N // ZB
    assert EPW * NW == E and NCHUNK * B == EPW and B % 16 == 0
    assert NRB * ZB == N

    mesh = plsc.VectorSubcoreMesh(core_axis_name="c", subcore_axis_name="s",
                                  num_cores=NC, num_subcores=NS)

    @functools.partial(
        pl.kernel,
        out_type=jax.ShapeDtypeStruct((NC, N, C), jnp.float32),
        mesh=mesh,
        scratch_types=[
            pltpu.VMEM((B,), jnp.int32),
            pltpu.VMEM((B,), jnp.int32),
            pltpu.VMEM((B,), jnp.int32),
            pltpu.VMEM((B, C), jnp.float32),
            pltpu.VMEM((B, C), jnp.float32),
            pltpu.VMEM((ZB, C), jnp.float32),
            pltpu.VMEM_SHARED((N, C), jnp.float32),
            pltpu.SemaphoreType.DMA,
            pltpu.SemaphoreType.DMA,
        ],
    )
    def sc_edge_kernel(abc_hbm, v_hbm, kidx_hbm, qidx_hbm, ko_hbm, out_hbm,
                       kbuf, qbuf, abuf, arows, vrows, zbuf, acc,
                       sem_a, sem_v):
        cid = lax.axis_index("c")
        sid = lax.axis_index("s")
        wid = cid * NS + sid

        zeros16 = jnp.zeros((16,), jnp.float32)
        for r in range(ZB):
            for j in range(C // 16):
                zbuf[r, pl.ds(j * 16, 16)] = zeros16

        @pl.loop(0, (NRB + NS - 1) // NS)
        def _zero(i):
            b = sid + i * NS

            @pl.when(b < NRB)
            def _():
                pltpu.sync_copy(zbuf, acc.at[pl.ds(b * ZB, ZB)])

        plsc.subcore_barrier()

        @pl.loop(0, NCHUNK)
        def _chunk(c):
            base = wid * EPW + c * B
            pltpu.sync_copy(kidx_hbm.at[pl.ds(base, B)], kbuf)
            pltpu.sync_copy(qidx_hbm.at[pl.ds(base, B)], qbuf)
            pltpu.sync_copy(ko_hbm.at[pl.ds(base, B)], abuf)
            for i in range(B // 16):
                sl = pl.ds(i * 16, 16)
                abuf[sl] = qbuf[sl] * KV + abuf[sl]
            cp_a = pltpu.async_copy(abc_hbm.at[abuf], arows, sem_a)
            cp_v = pltpu.async_copy(v_hbm.at[kbuf], vrows, sem_v)
            cp_a.wait()
            cp_v.wait()

            @pl.loop(0, B)
            def _mul(e):
                for j in range(C // 16):
                    sl = pl.ds(j * 16, 16)
                    vrows[e, sl] = vrows[e, sl] * arows[e, sl]

            pltpu.sync_copy(vrows, acc.at[qbuf], add=True)

        plsc.subcore_barrier()

        @pl.loop(0, (NRB + NS - 1) // NS)
        def _copy_out(i):
            b = sid + i * NS

            @pl.when(b < NRB)
            def _():
                pltpu.sync_copy(acc.at[pl.ds(b * ZB, ZB)],
                                out_hbm.at[cid, pl.ds(b * ZB, ZB)])

    return sc_edge_kernel


def _epilogue_body(parts_ref, Wo, bo, out_ref):
    s = parts_ref[0, :, :] + parts_ref[1, :, :]
    out_ref[...] = jnp.dot(s, Wo[...], preferred_element_type=jnp.float32) \
        + bo[...][None, :]


def kernel(feats, norm_points, kq_indices, kernel_offsets, Wp1, g1, b1,
           Wp2, g2, b2, Wp3, bp3, Wq, bq, Wv, bv, Wo, bo, pos_enc):
    N, C = feats.shape
    E = kernel_offsets.shape[0]
    KV, H, AH = pos_enc.shape

    m1, v1s, m2, v2s, P = pl.pallas_call(
        _stats_body,
        out_shape=(
            jax.ShapeDtypeStruct((3,), jnp.float32),
            jax.ShapeDtypeStruct((3,), jnp.float32),
            jax.ShapeDtypeStruct((C,), jnp.float32),
            jax.ShapeDtypeStruct((C,), jnp.float32),
            jax.ShapeDtypeStruct((C, KV * C), jnp.float32),
        ),
    )(norm_points, Wp1, g1, b1, Wp2, pos_enc)

    NBLK = 10
    BLK = N // NBLK
    full = lambda i: (0, 0)
    fullv = lambda i: (0,)
    blk = lambda i: (i, 0)
    a_bc, v = pl.pallas_call(
        _block_body,
        grid=(NBLK,),
        in_specs=[
            pl.BlockSpec((BLK, 3), blk),
            pl.BlockSpec((BLK, C), blk),
            pl.BlockSpec((3,), fullv),
            pl.BlockSpec((3,), fullv),
            pl.BlockSpec((3,), fullv),
            pl.BlockSpec((3,), fullv),
            pl.BlockSpec((C,), fullv),
            pl.BlockSpec((C,), fullv),
            pl.BlockSpec((C,), fullv),
            pl.BlockSpec((C,), fullv),
            pl.BlockSpec((3, 3), full),
            pl.BlockSpec((3, C), full),
            pl.BlockSpec((C, C), full),
            pl.BlockSpec((C,), fullv),
            pl.BlockSpec((C, C), full),
            pl.BlockSpec((C,), fullv),
            pl.BlockSpec((C, C), full),
            pl.BlockSpec((C,), fullv),
            pl.BlockSpec((C, KV * C), full),
        ],
        out_specs=(
            pl.BlockSpec((BLK, KV * C), blk),
            pl.BlockSpec((BLK, C), blk),
        ),
        out_shape=(
            jax.ShapeDtypeStruct((N, KV * C), jnp.float32),
            jax.ShapeDtypeStruct((N, C), jnp.float32),
        ),
    )(norm_points, feats, m1, v1s, g1, b1, m2, v2s, g2, b2,
      Wp1, Wp2, Wp3, bp3, Wq, bq, Wv, bv, P)
    a_bc = a_bc.reshape(N * KV, C)

    sc_edge = _make_sc_edge_kernel(N, E, C, KV)
    parts = sc_edge(a_bc, v, kq_indices[0], kq_indices[1], kernel_offsets)

    out = pl.pallas_call(
        _epilogue_body,
        out_shape=jax.ShapeDtypeStruct((N, C), jnp.float32),
    )(parts, Wo, bo)
    return out


# speedup vs baseline: 79.8508x; 1.2130x over previous
import functools

import jax
import jax.numpy as jnp
from jax import lax
from jax.experimental import pallas as pl
from jax.experimental.pallas import tpu as pltpu
from jax.experimental.pallas import tpu_sc as plsc


def _mlp_t1(pts, w1):
    return (pts[:, 0:1] * w1[0, :][None, :]
            + pts[:, 1:2] * w1[1, :][None, :]
            + pts[:, 2:3] * w1[2, :][None, :])


def _mlp_t2(t, w2):
    return (t[:, 0:1] * w2[0, :][None, :]
            + t[:, 1:2] * w2[1, :][None, :]
            + t[:, 2:3] * w2[2, :][None, :])


def _stats_body(np_ref, Wp1, g1, b1, Wp2, pe_ref,
                m1_out, v1_out, m2_out, v2_out, P_out):
    pts = np_ref[...]
    t = _mlp_t1(pts, Wp1[...])
    m = jnp.mean(t, axis=0)
    var = jnp.mean((t - m[None, :]) ** 2, axis=0)
    m1_out[...] = m
    v1_out[...] = var
    t = (t - m[None, :]) / jnp.sqrt(var[None, :] + 1e-5) * g1[...][None, :] \
        + b1[...][None, :]
    t = jnp.maximum(t, 0.0)
    t2 = _mlp_t2(t, Wp2[...])
    m2 = jnp.mean(t2, axis=0)
    m2_out[...] = m2
    v2_out[...] = jnp.mean((t2 - m2[None, :]) ** 2, axis=0)
    pe = pe_ref[...]
    kv, _, ah = pe.shape
    c_ = pe.shape[1] * ah
    pn = jnp.sqrt(jnp.sum(pe * pe, axis=-1, keepdims=True))
    npe = pe / jnp.maximum(pn, 1e-12)
    pe_flat = npe.reshape(kv, c_)
    pe_pad = jnp.concatenate([pe_flat, jnp.zeros((1, c_), jnp.float32)], 0)
    jj = lax.broadcasted_iota(jnp.int32, (c_, c_), 0) // ah
    mm = lax.broadcasted_iota(jnp.int32, (c_, c_), 1) // ah
    mask = (jj == mm).astype(jnp.float32)
    P_out[...] = pe_pad[:, :, None] * mask[None, :, :]


def _qv_body(np_ref, feats_ref, m1, v1, g1, b1, m2, v2, g2, b2,
             Wp1, Wp2, Wp3, bp3, Wq, bq, Wv, bv,
             nq_out, v_out):
    t = _mlp_t1(np_ref[...], Wp1[...])
    t = (t - m1[...][None, :]) / jnp.sqrt(v1[...][None, :] + 1e-5) \
        * g1[...][None, :] + b1[...][None, :]
    t = jnp.maximum(t, 0.0)
    t2 = _mlp_t2(t, Wp2[...])
    t2 = (t2 - m2[...][None, :]) / jnp.sqrt(v2[...][None, :] + 1e-5) \
        * g2[...][None, :] + b2[...][None, :]
    t2 = jnp.maximum(t2, 0.0)
    intra = jnp.dot(t2, Wp3[...], preferred_element_type=jnp.float32,
                    precision=jax.lax.Precision.HIGHEST) + bp3[...][None, :]
    x = feats_ref[...] + intra
    q = jnp.dot(x, Wq[...], preferred_element_type=jnp.float32,
                precision=jax.lax.Precision.HIGHEST) + bq[...][None, :]
    v_out[...] = jnp.dot(x, Wv[...], preferred_element_type=jnp.float32,
                         precision=jax.lax.Precision.HIGHEST) \
        + bv[...][None, :]
    n_, c_ = q.shape
    ah = 16
    q3 = q.reshape(n_, c_ // ah, ah)
    qn = jnp.sqrt(jnp.sum(q3 * q3, axis=-1, keepdims=True))
    nq_out[...] = (q3 / jnp.maximum(qn, 1e-12)).reshape(n_, c_)


def _atable_body(nq_ref, P_ref, out_ref):
    out_ref[0, :, :] = jnp.dot(
        nq_ref[...], P_ref[0, :, :], preferred_element_type=jnp.float32,
        precision=jax.lax.Precision.HIGHEST)


def _make_sc_edge_kernel(N, E, C, KV):
    NC, NS = 2, 16
    NW = NC * NS
    EPW = E // NW
    B = 48
    NCHUNK = EPW // B
    ZB = 16
    NRB = N // ZB
    assert EPW * NW == E and NCHUNK * B == EPW and B % 16 == 0
    assert NRB * ZB == N and NCHUNK % 2 == 1

    mesh = plsc.VectorSubcoreMesh(core_axis_name="c", subcore_axis_name="s",
                                  num_cores=NC, num_subcores=NS)

    @functools.partial(
        pl.kernel,
        out_type=jax.ShapeDtypeStruct((NC, N, C), jnp.float32),
        mesh=mesh,
        scratch_types=[
            pltpu.VMEM((3, B), jnp.int32),
            pltpu.VMEM((3, B), jnp.int32),
            pltpu.VMEM((B, C), jnp.float32),
            pltpu.VMEM((B, C), jnp.float32),
            pltpu.VMEM((B, C), jnp.float32),
            pltpu.VMEM((B, C), jnp.float32),
            pltpu.VMEM((B,), jnp.int32),
            pltpu.VMEM((B,), jnp.int32),
            pltpu.VMEM_SHARED((N, C), jnp.float32),
            pltpu.SemaphoreType.DMA,
            pltpu.SemaphoreType.DMA,
            pltpu.SemaphoreType.DMA,
            pltpu.SemaphoreType.DMA,
            pltpu.SemaphoreType.DMA,
            pltpu.SemaphoreType.DMA,
            pltpu.SemaphoreType.DMA,
            pltpu.SemaphoreType.DMA,
        ],
    )
    def sc_edge_kernel(abc_hbm, v_hbm, idx_hbm, out_hbm,
                       idx0, idx1, arows0, vrows0, arows1, vrows1,
                       qsc0, qsc1, acc,
                       si0, si1, sa0, sv0, sa1, sv1, ss0, ss1):
        cid = lax.axis_index("c")
        sid = lax.axis_index("s")
        wid = cid * NS + sid
        islots = ((idx0, si0), (idx1, si1))
        slots = ((arows0, vrows0, qsc0, sa0, sv0, ss0),
                 (arows1, vrows1, qsc1, sa1, sv1, ss1))

        @plsc.parallel_loop(0, ZB)
        def _zrow(r):
            for j in range(C // 16):
                vrows0[r, pl.ds(j * 16, 16)] = jnp.zeros((16,), jnp.float32)

        @pl.loop(0, (NRB + NS - 1) // NS)
        def _zero(i):
            b = sid + i * NS

            @pl.when(b < NRB)
            def _():
                pltpu.sync_copy(vrows0.at[pl.ds(0, ZB)],
                                acc.at[pl.ds(b * ZB, ZB)])

        plsc.subcore_barrier()

        def idx_start(c, islot):
            ib, sem = islots[islot]
            pltpu.async_copy(idx_hbm.at[wid, c], ib, sem)

        def idx_finish(c, islot):
            ib, sem = islots[islot]
            pltpu.make_async_copy(idx_hbm.at[wid, c], ib, sem).wait()
            for i in range(B // 16):
                sl = pl.ds(i * 16, 16)
                ib[2, sl] = ib[2, sl] * N + ib[1, sl]

        def fire(islot, slot):
            ib, _ = islots[islot]
            ar, vr, _, sa, sv, _ = slots[slot]
            pltpu.async_copy(abc_hbm.at[ib.at[2]], ar, sa)
            pltpu.async_copy(v_hbm.at[ib.at[0]], vr, sv)

        def process(islot, slot):
            ib, _ = islots[islot]
            ar, vr, qsc, sa, sv, ss = slots[slot]
            pltpu.make_async_copy(abc_hbm.at[ib.at[2]], ar, sa).wait()
            pltpu.make_async_copy(v_hbm.at[ib.at[0]], vr, sv).wait()

            @plsc.parallel_loop(0, B)
            def _mul(e):
                for j in range(C // 16):
                    sl = pl.ds(j * 16, 16)
                    vr[e, sl] = vr[e, sl] * ar[e, sl]

            for i in range(B // 16):
                sl = pl.ds(i * 16, 16)
                qsc[sl] = ib[1, sl]
            pltpu.async_copy(vr, acc.at[qsc], ss, add=True)

        def scatter_wait(slot):
            _, vr, qsc, _, _, ss = slots[slot]
            pltpu.make_async_copy(vr, acc.at[qsc], ss).wait()

        idx_start(0, 0)
        idx_finish(0, 0)
        fire(0, 0)
        idx_start(1, 1)

        @pl.loop(0, NCHUNK - 1, step=2)
        def _chunk(c):
            idx_finish(c + 1, 1)

            @pl.when(c > 0)
            def _():
                scatter_wait(1)

            fire(1, 1)
            process(0, 0)
            idx_start(c + 2, 0)
            idx_finish(c + 2, 0)
            process(1, 1)
            scatter_wait(0)
            fire(0, 0)

            @pl.when(c + 3 < NCHUNK)
            def _():
                idx_start(c + 3, 1)

        scatter_wait(1)
        process(0, 0)
        scatter_wait(0)

        plsc.subcore_barrier()

        @pl.loop(0, (NRB + NS - 1) // NS)
        def _copy_out(i):
            b = sid + i * NS

            @pl.when(b < NRB)
            def _():
                pltpu.sync_copy(acc.at[pl.ds(b * ZB, ZB)],
                                out_hbm.at[cid, pl.ds(b * ZB, ZB)])

    return sc_edge_kernel


def _epilogue_body(parts_ref, Wo, bo, out_ref):
    s = parts_ref[0, :, :] + parts_ref[1, :, :]
    out_ref[...] = jnp.dot(s, Wo[...], preferred_element_type=jnp.float32,
                           precision=jax.lax.Precision.HIGHEST) \
        + bo[...][None, :]


def kernel(feats, norm_points, kq_indices, kernel_offsets, Wp1, g1, b1,
           Wp2, g2, b2, Wp3, bp3, Wq, bq, Wv, bv, Wo, bo, pos_enc):
    N, C = feats.shape
    E = kernel_offsets.shape[0]
    KV, H, AH = pos_enc.shape

    m1, v1s, m2, v2s, P = pl.pallas_call(
        _stats_body,
        out_shape=(
            jax.ShapeDtypeStruct((3,), jnp.float32),
            jax.ShapeDtypeStruct((3,), jnp.float32),
            jax.ShapeDtypeStruct((C,), jnp.float32),
            jax.ShapeDtypeStruct((C,), jnp.float32),
            jax.ShapeDtypeStruct((KV + 1, C, C), jnp.float32),
        ),
    )(norm_points, Wp1, g1, b1, Wp2, pos_enc)

    NBLK = 5
    BLK = N // NBLK
    full = lambda i: (0, 0)
    fullv = lambda i: (0,)
    blk = lambda i: (i, 0)
    nq, v = pl.pallas_call(
        _qv_body,
        grid=(NBLK,),
        in_specs=[
            pl.BlockSpec((BLK, 3), blk),
            pl.BlockSpec((BLK, C), blk),
            pl.BlockSpec((3,), fullv),
            pl.BlockSpec((3,), fullv),
            pl.BlockSpec((3,), fullv),
            pl.BlockSpec((3,), fullv),
            pl.BlockSpec((C,), fullv),
            pl.BlockSpec((C,), fullv),
            pl.BlockSpec((C,), fullv),
            pl.BlockSpec((C,), fullv),
            pl.BlockSpec((3, 3), full),
            pl.BlockSpec((3, C), full),
            pl.BlockSpec((C, C), full),
            pl.BlockSpec((C,), fullv),
            pl.BlockSpec((C, C), full),
            pl.BlockSpec((C,), fullv),
            pl.BlockSpec((C, C), full),
            pl.BlockSpec((C,), fullv),
        ],
        out_specs=(
            pl.BlockSpec((BLK, C), blk),
            pl.BlockSpec((BLK, C), blk),
        ),
        out_shape=(
            jax.ShapeDtypeStruct((N, C), jnp.float32),
            jax.ShapeDtypeStruct((N, C), jnp.float32),
        ),
    )(norm_points, feats, m1, v1s, g1, b1, m2, v2s, g2, b2,
      Wp1, Wp2, Wp3, bp3, Wq, bq, Wv, bv)

    a_bc = pl.pallas_call(
        _atable_body,
        grid=(NBLK, KV + 1),
        in_specs=[
            pl.BlockSpec((BLK, C), lambda i, o: (i, 0)),
            pl.BlockSpec((1, C, C), lambda i, o: (o, 0, 0)),
        ],
        out_specs=pl.BlockSpec((1, BLK, C), lambda i, o: (o, i, 0)),
        out_shape=jax.ShapeDtypeStruct((KV + 1, N, C), jnp.float32),
    )(nq, P)
    a_bc = a_bc.reshape((KV + 1) * N, C)

    NW = 32
    B = 48
    NCHUNK = -(-(E // NW) // B)
    if NCHUNK % 2 == 0:
        NCHUNK += 1
    EPW = NCHUNK * B
    EP = EPW * NW
    pad = EP - E
    k_pad = jnp.concatenate([kq_indices[0], jnp.zeros((pad,), jnp.int32)])
    q_pad = jnp.concatenate([kq_indices[1], jnp.zeros((pad,), jnp.int32)])
    o_pad = jnp.concatenate([kernel_offsets,
                             jnp.full((pad,), KV, jnp.int32)])
    idx = jnp.stack([
        k_pad.reshape(NW, NCHUNK, B),
        q_pad.reshape(NW, NCHUNK, B),
        o_pad.reshape(NW, NCHUNK, B),
    ], axis=2)

    sc_edge = _make_sc_edge_kernel(N, EP, C, KV)
    parts = sc_edge(a_bc, v, idx)

    out = pl.pallas_call(
        _epilogue_body,
        out_shape=jax.ShapeDtypeStruct((N, C), jnp.float32),
    )(parts, Wo, bo)
    return out


# speedup vs baseline: 86.2463x; 1.0801x over previous
import functools

import jax
import jax.numpy as jnp
from jax import lax
from jax.experimental import pallas as pl
from jax.experimental.pallas import tpu as pltpu
from jax.experimental.pallas import tpu_sc as plsc


def _mlp_t1(pts, w1):
    return (pts[:, 0:1] * w1[0, :][None, :]
            + pts[:, 1:2] * w1[1, :][None, :]
            + pts[:, 2:3] * w1[2, :][None, :])


def _mlp_t2(t, w2):
    return (t[:, 0:1] * w2[0, :][None, :]
            + t[:, 1:2] * w2[1, :][None, :]
            + t[:, 2:3] * w2[2, :][None, :])


def _stats_body(np_ref, Wp1, g1, b1, Wp2, pe_ref,
                m1_out, v1_out, m2_out, v2_out, P_out):
    pts = np_ref[...]
    t = _mlp_t1(pts, Wp1[...])
    m = jnp.mean(t, axis=0)
    var = jnp.mean((t - m[None, :]) ** 2, axis=0)
    m1_out[...] = m
    v1_out[...] = var
    t = (t - m[None, :]) / jnp.sqrt(var[None, :] + 1e-5) * g1[...][None, :] \
        + b1[...][None, :]
    t = jnp.maximum(t, 0.0)
    t2 = _mlp_t2(t, Wp2[...])
    m2 = jnp.mean(t2, axis=0)
    m2_out[...] = m2
    v2_out[...] = jnp.mean((t2 - m2[None, :]) ** 2, axis=0)
    pe = pe_ref[...]
    kv, _, ah = pe.shape
    c_ = pe.shape[1] * ah
    pn = jnp.sqrt(jnp.sum(pe * pe, axis=-1, keepdims=True))
    npe = pe / jnp.maximum(pn, 1e-12)
    pe_flat = npe.reshape(kv, c_)
    pe_pad = jnp.concatenate([pe_flat, jnp.zeros((1, c_), jnp.float32)], 0)
    jj = lax.broadcasted_iota(jnp.int32, (c_, c_), 0) // ah
    mm = lax.broadcasted_iota(jnp.int32, (c_, c_), 1) // ah
    mask = (jj == mm).astype(jnp.float32)
    P_out[...] = pe_pad[:, :, None] * mask[None, :, :]


def _qv_body(np_ref, feats_ref, m1, v1, g1, b1, m2, v2, g2, b2,
             Wp1, Wp2, Wp3, bp3, Wq, bq, Wv, bv,
             nq_out, v_out):
    t = _mlp_t1(np_ref[...], Wp1[...])
    t = (t - m1[...][None, :]) / jnp.sqrt(v1[...][None, :] + 1e-5) \
        * g1[...][None, :] + b1[...][None, :]
    t = jnp.maximum(t, 0.0)
    t2 = _mlp_t2(t, Wp2[...])
    t2 = (t2 - m2[...][None, :]) / jnp.sqrt(v2[...][None, :] + 1e-5) \
        * g2[...][None, :] + b2[...][None, :]
    t2 = jnp.maximum(t2, 0.0)
    intra = jnp.dot(t2, Wp3[...], preferred_element_type=jnp.float32) + bp3[...][None, :]
    x = feats_ref[...] + intra
    q = jnp.dot(x, Wq[...], preferred_element_type=jnp.float32) + bq[...][None, :]
    v_out[...] = jnp.dot(x, Wv[...], preferred_element_type=jnp.float32) \
        + bv[...][None, :]
    n_, c_ = q.shape
    ah = 16
    q3 = q.reshape(n_, c_ // ah, ah)
    qn = jnp.sqrt(jnp.sum(q3 * q3, axis=-1, keepdims=True))
    nq_out[...] = (q3 / jnp.maximum(qn, 1e-12)).reshape(n_, c_)


def _atable_body(nq_ref, P_ref, out_ref):
    out_ref[0, :, :] = jnp.dot(
        nq_ref[...], P_ref[0, :, :], preferred_element_type=jnp.float32)


def _make_sc_edge_kernel(N, E, C, KV):
    NC, NS = 2, 16
    NW = NC * NS
    EPW = E // NW
    B = 48
    NCHUNK = EPW // B
    ZB = 16
    NRB = N // ZB
    assert EPW * NW == E and NCHUNK * B == EPW and B % 16 == 0
    assert NRB * ZB == N and NCHUNK % 2 == 1

    mesh = plsc.VectorSubcoreMesh(core_axis_name="c", subcore_axis_name="s",
                                  num_cores=NC, num_subcores=NS)

    @functools.partial(
        pl.kernel,
        out_type=jax.ShapeDtypeStruct((NC, N, C), jnp.float32),
        mesh=mesh,
        scratch_types=[
            pltpu.VMEM((3, B), jnp.int32),
            pltpu.VMEM((3, B), jnp.int32),
            pltpu.VMEM((B, C), jnp.float32),
            pltpu.VMEM((B, C), jnp.float32),
            pltpu.VMEM((B, C), jnp.float32),
            pltpu.VMEM((B, C), jnp.float32),
            pltpu.VMEM((B,), jnp.int32),
            pltpu.VMEM((B,), jnp.int32),
            pltpu.VMEM_SHARED((N, C), jnp.float32),
            pltpu.SemaphoreType.DMA,
            pltpu.SemaphoreType.DMA,
            pltpu.SemaphoreType.DMA,
            pltpu.SemaphoreType.DMA,
            pltpu.SemaphoreType.DMA,
            pltpu.SemaphoreType.DMA,
            pltpu.SemaphoreType.DMA,
            pltpu.SemaphoreType.DMA,
        ],
    )
    def sc_edge_kernel(abc_hbm, v_hbm, idx_hbm, out_hbm,
                       idx0, idx1, arows0, vrows0, arows1, vrows1,
                       qsc0, qsc1, acc,
                       si0, si1, sa0, sv0, sa1, sv1, ss0, ss1):
        cid = lax.axis_index("c")
        sid = lax.axis_index("s")
        wid = cid * NS + sid
        islots = ((idx0, si0), (idx1, si1))
        slots = ((arows0, vrows0, qsc0, sa0, sv0, ss0),
                 (arows1, vrows1, qsc1, sa1, sv1, ss1))

        @plsc.parallel_loop(0, ZB)
        def _zrow(r):
            for j in range(C // 16):
                vrows0[r, pl.ds(j * 16, 16)] = jnp.zeros((16,), jnp.float32)

        @pl.loop(0, (NRB + NS - 1) // NS)
        def _zero(i):
            b = sid + i * NS

            @pl.when(b < NRB)
            def _():
                pltpu.sync_copy(vrows0.at[pl.ds(0, ZB)],
                                acc.at[pl.ds(b * ZB, ZB)])

        plsc.subcore_barrier()

        def idx_start(c, islot):
            ib, sem = islots[islot]
            pltpu.async_copy(idx_hbm.at[wid, c], ib, sem)

        def idx_finish(c, islot):
            ib, sem = islots[islot]
            pltpu.make_async_copy(idx_hbm.at[wid, c], ib, sem).wait()
            for i in range(B // 16):
                sl = pl.ds(i * 16, 16)
                ib[2, sl] = ib[2, sl] * N + ib[1, sl]

        def fire(islot, slot):
            ib, _ = islots[islot]
            ar, vr, _, sa, sv, _ = slots[slot]
            pltpu.async_copy(abc_hbm.at[ib.at[2]], ar, sa)
            pltpu.async_copy(v_hbm.at[ib.at[0]], vr, sv)

        def process(islot, slot):
            ib, _ = islots[islot]
            ar, vr, qsc, sa, sv, ss = slots[slot]
            pltpu.make_async_copy(abc_hbm.at[ib.at[2]], ar, sa).wait()
            pltpu.make_async_copy(v_hbm.at[ib.at[0]], vr, sv).wait()

            @plsc.parallel_loop(0, B)
            def _mul(e):
                for j in range(C // 16):
                    sl = pl.ds(j * 16, 16)
                    vr[e, sl] = vr[e, sl] * ar[e, sl]

            for i in range(B // 16):
                sl = pl.ds(i * 16, 16)
                qsc[sl] = ib[1, sl]
            pltpu.async_copy(vr, acc.at[qsc], ss, add=True)

        def scatter_wait(slot):
            _, vr, qsc, _, _, ss = slots[slot]
            pltpu.make_async_copy(vr, acc.at[qsc], ss).wait()

        idx_start(0, 0)
        idx_finish(0, 0)
        fire(0, 0)
        idx_start(1, 1)

        @pl.loop(0, NCHUNK - 1, step=2)
        def _chunk(c):
            idx_finish(c + 1, 1)

            @pl.when(c > 0)
            def _():
                scatter_wait(1)

            fire(1, 1)
            process(0, 0)
            idx_start(c + 2, 0)
            idx_finish(c + 2, 0)
            process(1, 1)
            scatter_wait(0)
            fire(0, 0)

            @pl.when(c + 3 < NCHUNK)
            def _():
                idx_start(c + 3, 1)

        scatter_wait(1)
        process(0, 0)
        scatter_wait(0)

        plsc.subcore_barrier()

        @pl.loop(0, (NRB + NS - 1) // NS)
        def _copy_out(i):
            b = sid + i * NS

            @pl.when(b < NRB)
            def _():
                pltpu.sync_copy(acc.at[pl.ds(b * ZB, ZB)],
                                out_hbm.at[cid, pl.ds(b * ZB, ZB)])

    return sc_edge_kernel


def _epilogue_body(parts_ref, Wo, bo, out_ref):
    s = parts_ref[0, :, :] + parts_ref[1, :, :]
    out_ref[...] = jnp.dot(s, Wo[...], preferred_element_type=jnp.float32) \
        + bo[...][None, :]


def kernel(feats, norm_points, kq_indices, kernel_offsets, Wp1, g1, b1,
           Wp2, g2, b2, Wp3, bp3, Wq, bq, Wv, bv, Wo, bo, pos_enc):
    N, C = feats.shape
    E = kernel_offsets.shape[0]
    KV, H, AH = pos_enc.shape

    m1, v1s, m2, v2s, P = pl.pallas_call(
        _stats_body,
        out_shape=(
            jax.ShapeDtypeStruct((3,), jnp.float32),
            jax.ShapeDtypeStruct((3,), jnp.float32),
            jax.ShapeDtypeStruct((C,), jnp.float32),
            jax.ShapeDtypeStruct((C,), jnp.float32),
            jax.ShapeDtypeStruct((KV + 1, C, C), jnp.float32),
        ),
    )(norm_points, Wp1, g1, b1, Wp2, pos_enc)

    NBLK = 5
    BLK = N // NBLK
    full = lambda i: (0, 0)
    fullv = lambda i: (0,)
    blk = lambda i: (i, 0)
    nq, v = pl.pallas_call(
        _qv_body,
        grid=(NBLK,),
        in_specs=[
            pl.BlockSpec((BLK, 3), blk),
            pl.BlockSpec((BLK, C), blk),
            pl.BlockSpec((3,), fullv),
            pl.BlockSpec((3,), fullv),
            pl.BlockSpec((3,), fullv),
            pl.BlockSpec((3,), fullv),
            pl.BlockSpec((C,), fullv),
            pl.BlockSpec((C,), fullv),
            pl.BlockSpec((C,), fullv),
            pl.BlockSpec((C,), fullv),
            pl.BlockSpec((3, 3), full),
            pl.BlockSpec((3, C), full),
            pl.BlockSpec((C, C), full),
            pl.BlockSpec((C,), fullv),
            pl.BlockSpec((C, C), full),
            pl.BlockSpec((C,), fullv),
            pl.BlockSpec((C, C), full),
            pl.BlockSpec((C,), fullv),
        ],
        out_specs=(
            pl.BlockSpec((BLK, C), blk),
            pl.BlockSpec((BLK, C), blk),
        ),
        out_shape=(
            jax.ShapeDtypeStruct((N, C), jnp.float32),
            jax.ShapeDtypeStruct((N, C), jnp.float32),
        ),
    )(norm_points, feats, m1, v1s, g1, b1, m2, v2s, g2, b2,
      Wp1, Wp2, Wp3, bp3, Wq, bq, Wv, bv)

    a_bc = pl.pallas_call(
        _atable_body,
        grid=(NBLK, KV + 1),
        in_specs=[
            pl.BlockSpec((BLK, C), lambda i, o: (i, 0)),
            pl.BlockSpec((1, C, C), lambda i, o: (o, 0, 0)),
        ],
        out_specs=pl.BlockSpec((1, BLK, C), lambda i, o: (o, i, 0)),
        out_shape=jax.ShapeDtypeStruct((KV + 1, N, C), jnp.float32),
    )(nq, P)
    a_bc = a_bc.reshape((KV + 1) * N, C)

    NW = 32
    B = 48
    NCHUNK = -(-(E // NW) // B)
    if NCHUNK % 2 == 0:
        NCHUNK += 1
    EPW = NCHUNK * B
    EP = EPW * NW
    pad = EP - E
    k_pad = jnp.concatenate([kq_indices[0], jnp.zeros((pad,), jnp.int32)])
    q_pad = jnp.concatenate([kq_indices[1], jnp.zeros((pad,), jnp.int32)])
    o_pad = jnp.concatenate([kernel_offsets,
                             jnp.full((pad,), KV, jnp.int32)])
    idx = jnp.stack([
        k_pad.reshape(NW, NCHUNK, B),
        q_pad.reshape(NW, NCHUNK, B),
        o_pad.reshape(NW, NCHUNK, B),
    ], axis=2)

    sc_edge = _make_sc_edge_kernel(N, EP, C, KV)
    parts = sc_edge(a_bc, v, idx)

    out = pl.pallas_call(
        _epilogue_body,
        out_shape=jax.ShapeDtypeStruct((N, C), jnp.float32),
    )(parts, Wo, bo)
    return out
